# baseline (speedup 1.0000x reference)
import functools

import jax
import jax.numpy as jnp
from jax.experimental import pallas as pl


def _moe_slice(xb, wg, we_cat, e):
    nv = xb.shape[1]

    logits = jnp.dot(wg, xb, preferred_element_type=jnp.float32)
    m = jnp.max(logits, axis=0, keepdims=True)
    ex = jnp.exp(logits - m)
    g = ex / jnp.sum(ex, axis=0, keepdims=True)

    neg = jnp.float32(-jnp.inf)
    m1 = jnp.max(g, axis=0, keepdims=True)
    m2 = jnp.max(jnp.where(g < m1, g, neg), axis=0, keepdims=True)
    wd = jnp.where(g >= m2, g, 0.0)

    xb_b = xb.astype(jnp.bfloat16)
    wd_b = wd.astype(jnp.bfloat16)
    xw = jnp.concatenate(
        [wd_b[i : i + 1, :] * xb_b for i in range(e)] + [wd_b], axis=0
    )
    out = jnp.dot(we_cat, xw, preferred_element_type=jnp.float32)
    return out, g


def _moe_body(x_ref, wg_ref, we_ref, out_ref, gate_ref, *, nsteps, bt, e, nh):
    s = pl.program_id(0)
    h = pl.program_id(1)
    wg = wg_ref[...]
    we_cat = we_ref[...]
    bth = bt // nh

    gsum = None
    for bi in range(bth):
        out, g = _moe_slice(x_ref[h * bth + bi], wg, we_cat, e)
        out_ref[bi] = out
        gsum = g if gsum is None else gsum + g

    @pl.when((s == 0) & (h == 0))
    def _init():
        gate_ref[...] = jnp.zeros_like(gate_ref)

    gate_ref[...] += gsum

    @pl.when((s == nsteps - 1) & (h == nh - 1))
    def _fin():
        gate_ref[...] = gate_ref[...] * (1.0 / (nsteps * bt))


@jax.jit
def kernel(x, Wg, We, be):
    B, IN_LEN, NVARS = x.shape
    E, OUT_LEN, _ = We.shape
    BT = 8
    nsteps = B // BT

    we_cat = jnp.concatenate(
        [We.transpose(1, 0, 2).reshape(OUT_LEN, E * IN_LEN), be.T], axis=1
    ).astype(jnp.bfloat16)

    NH = 2
    body = functools.partial(_moe_body, nsteps=nsteps, bt=BT, e=E, nh=NH)
    out, gate_sum = pl.pallas_call(
        body,
        grid=(nsteps, NH),
        in_specs=[
            pl.BlockSpec((BT, IN_LEN, NVARS), lambda s, h: (s, 0, 0)),
            pl.BlockSpec((E, IN_LEN), lambda s, h: (0, 0)),
            pl.BlockSpec((OUT_LEN, E * IN_LEN + E), lambda s, h: (0, 0)),
        ],
        out_specs=[
            pl.BlockSpec((BT // NH, OUT_LEN, NVARS), lambda s, h: (s * NH + h, 0, 0)),
            pl.BlockSpec((E, NVARS), lambda s, h: (0, 0)),
        ],
        out_shape=[
            jax.ShapeDtypeStruct((B, OUT_LEN, NVARS), x.dtype),
            jax.ShapeDtypeStruct((E, NVARS), jnp.float32),
        ],
    )(x, Wg, we_cat)

    gate_mean = gate_sum.T
    return (out, gate_mean)


# speedup vs baseline: 1.0663x; 1.0663x over previous
import functools

import jax
import jax.numpy as jnp
from jax.experimental import pallas as pl
from jax.experimental.pallas import tpu as pltpu


def _moe_slice(xb, wg, we_cat, e):
    logits = jnp.dot(wg, xb, preferred_element_type=jnp.float32)
    m = jnp.max(logits, axis=0, keepdims=True)
    ex = jnp.exp(logits - m)
    g = ex / jnp.sum(ex, axis=0, keepdims=True)

    neg = jnp.float32(-jnp.inf)
    m1 = jnp.max(g, axis=0, keepdims=True)
    m2 = jnp.max(jnp.where(g < m1, g, neg), axis=0, keepdims=True)
    wd = jnp.where(g >= m2, g, 0.0)

    xb_b = xb.astype(jnp.bfloat16)
    wd_b = wd.astype(jnp.bfloat16)
    xw = jnp.concatenate(
        [wd_b[i : i + 1, :] * xb_b for i in range(e)] + [wd_b], axis=0
    )
    out = jnp.dot(we_cat, xw, preferred_element_type=jnp.float32)
    return out, g


def _moe_body(
    x_hbm,
    wg_ref,
    we_ref,
    out_hbm,
    gate_ref,
    xbuf,
    obuf,
    in_sems,
    out_sems,
    *,
    nsteps,
    bt,
    e,
    depth,
):
    s = pl.program_id(0)

    def in_copy(blk, slot):
        return pltpu.make_async_copy(
            x_hbm.at[pl.ds(blk * bt, bt)], xbuf.at[slot], in_sems.at[slot]
        )

    def out_copy(blk, slot):
        return pltpu.make_async_copy(
            obuf.at[slot], out_hbm.at[pl.ds(blk * bt, bt)], out_sems.at[slot]
        )

    @pl.when(s == 0)
    def _prologue():
        for d in range(depth):
            in_copy(d, d).start()

    slot_in = jax.lax.rem(s, depth)
    slot_out = jax.lax.rem(s, 2)

    in_copy(s, slot_in).wait()

    @pl.when(s >= 2)
    def _free_obuf():
        out_copy(s - 2, slot_out).wait()

    wg = wg_ref[...]
    we_cat = we_ref[...]
    gsum = None
    for bi in range(bt):
        out, g = _moe_slice(xbuf[slot_in, bi], wg, we_cat, e)
        obuf[slot_out, bi] = out
        gsum = g if gsum is None else gsum + g

    out_copy(s, slot_out).start()

    @pl.when(s + depth < nsteps)
    def _prefetch():
        in_copy(s + depth, slot_in).start()

    @pl.when(s == 0)
    def _init():
        gate_ref[...] = jnp.zeros_like(gate_ref)

    gate_ref[...] += gsum

    @pl.when(s == nsteps - 1)
    def _fin():
        gate_ref[...] = gate_ref[...] * (1.0 / (nsteps * bt))
        out_copy(s - 1, jax.lax.rem(s - 1, 2)).wait()
        out_copy(s, slot_out).wait()


@jax.jit
def kernel(x, Wg, We, be):
    B, IN_LEN, NVARS = x.shape
    E, OUT_LEN, _ = We.shape
    BT = 8
    DEPTH = 3
    nsteps = B // BT

    we_cat = jnp.concatenate(
        [We.transpose(1, 0, 2).reshape(OUT_LEN, E * IN_LEN), be.T], axis=1
    ).astype(jnp.bfloat16)

    body = functools.partial(_moe_body, nsteps=nsteps, bt=BT, e=E, depth=DEPTH)
    out, gate_sum = pl.pallas_call(
        body,
        grid=(nsteps,),
        in_specs=[
            pl.BlockSpec(memory_space=pl.ANY),
            pl.BlockSpec((E, IN_LEN), lambda s: (0, 0)),
            pl.BlockSpec((OUT_LEN, E * IN_LEN + E), lambda s: (0, 0)),
        ],
        out_specs=[
            pl.BlockSpec(memory_space=pl.ANY),
            pl.BlockSpec((E, NVARS), lambda s: (0, 0)),
        ],
        out_shape=[
            jax.ShapeDtypeStruct((B, OUT_LEN, NVARS), x.dtype),
            jax.ShapeDtypeStruct((E, NVARS), jnp.float32),
        ],
        scratch_shapes=[
            pltpu.VMEM((DEPTH, BT, IN_LEN, NVARS), jnp.float32),
            pltpu.VMEM((2, BT, OUT_LEN, NVARS), jnp.float32),
            pltpu.SemaphoreType.DMA((DEPTH,)),
            pltpu.SemaphoreType.DMA((2,)),
        ],
    )(x, Wg, we_cat)

    gate_mean = gate_sum.T
    return (out, gate_mean)


# speedup vs baseline: 1.1715x; 1.0987x over previous
import functools

import jax
import jax.numpy as jnp
from jax.experimental import pallas as pl


def _moe_slice(xb, wg, we_cat, e):
    nv = xb.shape[1]

    logits = jnp.dot(wg, xb, preferred_element_type=jnp.float32)
    m = jnp.max(logits, axis=0, keepdims=True)
    ex = jnp.exp(logits - m)
    g = ex / jnp.sum(ex, axis=0, keepdims=True)

    neg = jnp.float32(-jnp.inf)
    m1 = jnp.max(g, axis=0, keepdims=True)
    m2 = jnp.max(jnp.where(g < m1, g, neg), axis=0, keepdims=True)
    wd = jnp.where(g >= m2, g, 0.0)

    xb_b = xb.astype(jnp.bfloat16)
    wd_b = wd.astype(jnp.bfloat16)
    inlen = xb.shape[0]
    out = jnp.dot(we_cat[:, e * inlen :], wd_b, preferred_element_type=jnp.float32)
    for pp in range(e // 2):
        xw_p = jnp.concatenate(
            [wd_b[2 * pp : 2 * pp + 1, :] * xb_b,
             wd_b[2 * pp + 1 : 2 * pp + 2, :] * xb_b], axis=0
        )
        out = out + jnp.dot(
            we_cat[:, 2 * pp * inlen : (2 * pp + 2) * inlen],
            xw_p,
            preferred_element_type=jnp.float32,
        )
    return out, g


def _moe_body(x_ref, wg_ref, we_ref, out_ref, gate_ref, *, nsteps, bt, e):
    s = pl.program_id(0)
    wg = wg_ref[...]
    we_cat = we_ref[...]

    gsum = None
    for bi in range(bt):
        out, g = _moe_slice(x_ref[bi], wg, we_cat, e)
        out_ref[bi] = out
        gsum = g if gsum is None else gsum + g

    @pl.when(s == 0)
    def _init():
        gate_ref[...] = jnp.zeros_like(gate_ref)

    gate_ref[...] += gsum

    @pl.when(s == nsteps - 1)
    def _fin():
        gate_ref[...] = gate_ref[...] * (1.0 / (nsteps * bt))


@jax.jit
def kernel(x, Wg, We, be):
    B, IN_LEN, NVARS = x.shape
    E, OUT_LEN, _ = We.shape
    BT = 8
    nsteps = B // BT

    we_cat = jnp.concatenate(
        [We.transpose(1, 0, 2).reshape(OUT_LEN, E * IN_LEN), be.T], axis=1
    ).astype(jnp.bfloat16)

    body = functools.partial(_moe_body, nsteps=nsteps, bt=BT, e=E)
    out, gate_sum = pl.pallas_call(
        body,
        grid=(nsteps,),
        in_specs=[
            pl.BlockSpec((BT, IN_LEN, NVARS), lambda s: (s, 0, 0)),
            pl.BlockSpec((E, IN_LEN), lambda s: (0, 0)),
            pl.BlockSpec((OUT_LEN, E * IN_LEN + E), lambda s: (0, 0)),
        ],
        out_specs=[
            pl.BlockSpec((BT, OUT_LEN, NVARS), lambda s: (s, 0, 0)),
            pl.BlockSpec((E, NVARS), lambda s: (0, 0)),
        ],
        out_shape=[
            jax.ShapeDtypeStruct((B, OUT_LEN, NVARS), x.dtype),
            jax.ShapeDtypeStruct((E, NVARS), jnp.float32),
        ],
    )(x, Wg, we_cat)

    gate_mean = gate_sum.T
    return (out, gate_mean)
